# baseline (speedup 1.0000x reference)
import functools

import jax
import jax.numpy as jnp
from jax import lax
from jax.experimental import pallas as pl
from jax.experimental.pallas import tpu as pltpu
from jax.experimental.pallas import tpu_sc as plsc

NC, NS = 2, 16
NW = NC * NS
CB = 128
NBUF = 8
LAG = 4

_MESH = plsc.VectorSubcoreMesh(core_axis_name="c", subcore_axis_name="s")


def _wid():
    return lax.axis_index("s") * NC + lax.axis_index("c")


GROW = 320


@functools.partial(jax.jit, static_argnums=(1, 2))
def _compact(table, v, d):
    ng = v // GROW
    tail = v - ng * GROW

    @functools.partial(
        pl.kernel,
        out_type=jax.ShapeDtypeStruct((v * d // 128, d * 2), jnp.float32),
        mesh=_MESH,
        scratch_types=[
            pltpu.VMEM((2, GROW, d), jnp.float32),
            pltpu.VMEM((2, GROW // 2, d * 2), jnp.float32),
            pltpu.SemaphoreType.DMA((2,)),
            pltpu.SemaphoreType.DMA((2,)),
        ],
        compiler_params=pltpu.CompilerParams(use_tc_tiling_on_sc=True),
    )
    def k(tab_hbm, out_hbm, vin, vout, isem, osem):
        wid = _wid()

        def in_legs(g, slot):
            r0 = pl.multiple_of(g * GROW, 8)
            return tab_hbm.at[pl.ds(r0, GROW)], vin.at[slot]

        def out_legs(g, slot):
            r0 = pl.multiple_of(g * (GROW // 2), 8)
            return vout.at[slot], out_hbm.at[pl.ds(r0, GROW // 2)]

        def repack(slot):
            def mv(t, carry):
                r = t >> 2
                c16 = jnp.bitwise_and(t, 3) << 4
                x = vin[slot, r, pl.ds(c16, 16)]
                vout[slot, r >> 1,
                     pl.ds(jnp.bitwise_and(r, 1) * d + c16, 16)] = x
                return carry

            lax.fori_loop(0, GROW * (d // 16), mv, 0)

        def body(i, carry):
            g = wid + i * NW
            slot = jnp.bitwise_and(i, 1)

            @pl.when(i > 1)
            def _():
                s, o = out_legs(g - 2 * NW, slot)
                pltpu.make_async_copy(s, o, osem.at[slot]).wait()

            s, o = in_legs(g, slot)
            pltpu.async_copy(s, o, isem.at[slot])
            pltpu.make_async_copy(s, o, isem.at[slot]).wait()
            repack(slot)
            s, o = out_legs(g, slot)
            pltpu.async_copy(s, o, osem.at[slot])
            return carry

        nmine = (ng - wid + NW - 1) // NW
        lax.fori_loop(0, nmine, body, 0)

        def drain(i, carry):
            j = nmine - 2 + i
            g = wid + j * NW

            @pl.when((j >= 0) & (g < ng))
            def _():
                s, o = out_legs(g, jnp.bitwise_and(j, 1))
                pltpu.make_async_copy(s, o, osem.at[jnp.bitwise_and(j, 1)]
                                      ).wait()
            return carry

        lax.fori_loop(0, 2, drain, 0)

        if tail:
            @pl.when(wid == 0)
            def _():
                r0 = ng * GROW
                s = tab_hbm.at[pl.ds(pl.multiple_of(r0, 8), tail)]
                o = vin.at[0, pl.ds(0, tail)]
                pltpu.async_copy(s, o, isem.at[0])
                pltpu.make_async_copy(s, o, isem.at[0]).wait()

                def mv(t, carry):
                    r = t >> 2
                    c16 = jnp.bitwise_and(t, 3) << 4
                    x = vin[0, r, pl.ds(c16, 16)]
                    vout[0, r >> 1,
                         pl.ds(jnp.bitwise_and(r, 1) * d + c16, 16)] = x
                    return carry

                lax.fori_loop(0, tail * (d // 16), mv, 0)
                s2 = vout.at[0, pl.ds(0, tail // 2)]
                o2 = out_hbm.at[pl.ds(pl.multiple_of(r0 // 2, 8), tail // 2)]
                pltpu.async_copy(s2, o2, osem.at[0])
                pltpu.make_async_copy(s2, o2, osem.at[0]).wait()

    return k(table)


@functools.partial(jax.jit, static_argnums=(2, 3))
def _gather(idx_flat, table, nch, d):
    rpw = nch * CB

    @functools.partial(
        pl.kernel,
        out_type=jax.ShapeDtypeStruct((NW * rpw, d), jnp.float32),
        mesh=_MESH,
        scratch_types=[
            pltpu.VMEM((nch * CB,), jnp.int32),
            pltpu.VMEM((NBUF, CB, d), jnp.float32),
            pltpu.SemaphoreType.DMA((NBUF,)),
            pltpu.SemaphoreType.DMA((NBUF,)),
        ],
        compiler_params=pltpu.CompilerParams(use_tc_tiling_on_sc=False),
    )
    def k(idx_hbm, table_hbm, out_hbm, idx_v, rows_v, gsem, osem):
        wid = _wid()
        base = wid * rpw
        pltpu.sync_copy(idx_hbm.at[pl.ds(base, rpw)], idx_v)

        def start_gather(j, slot):
            pltpu.async_copy(
                table_hbm.at[idx_v.at[pl.ds(j * CB, CB)]], rows_v.at[slot],
                gsem.at[slot])

        def drain_chunk(jd, slot):
            pltpu.make_async_copy(
                table_hbm.at[idx_v.at[pl.ds(jd * CB, CB)]], rows_v.at[slot],
                gsem.at[slot]
            ).wait()
            pltpu.async_copy(
                rows_v.at[slot], out_hbm.at[pl.ds(base + jd * CB, CB)],
                osem.at[slot])

        def wait_out(jd, slot):
            pltpu.make_async_copy(
                rows_v.at[slot], out_hbm.at[pl.ds(base + jd * CB, CB)],
                osem.at[slot]
            ).wait()

        for j in range(NBUF):
            start_gather(j, j)
            if j >= LAG:
                drain_chunk(j - LAG, j - LAG)

        def body(j, carry):
            slot = jnp.bitwise_and(j, NBUF - 1)
            wait_out(j - NBUF, slot)
            start_gather(j, slot)
            jw = j - LAG
            drain_chunk(jw, jnp.bitwise_and(jw, NBUF - 1))
            return carry

        lax.fori_loop(NBUF, nch, body, 0)

        for jd in range(nch - LAG, nch):
            drain_chunk(jd, jd % NBUF)
        for jd in range(nch - NBUF, nch):
            wait_out(jd, jd % NBUF)

    return k(idx_flat, table)


GB = 8


@functools.partial(jax.jit, static_argnums=(1, 2, 3))
def _relayout(rows128, b, s, d):
    bpw = b // NW
    ngr = bpw // GB
    rows_g = GB * s * d // 128

    @functools.partial(
        pl.kernel,
        out_type=jax.ShapeDtypeStruct((b, s, d), jnp.float32),
        mesh=_MESH,
        scratch_types=[
            pltpu.VMEM((2, GB * s * d // 128, 128), jnp.float32),
            pltpu.VMEM((GB, s, d), jnp.float32),
            pltpu.SemaphoreType.DMA((2,)),
            pltpu.SemaphoreType.DMA,
        ],
        compiler_params=pltpu.CompilerParams(use_tc_tiling_on_sc=True),
    )
    def k(flat_hbm, out_hbm, vin, vout, isem, osem):
        wid = _wid()
        base = wid * bpw

        def in_legs(g, slot):
            r0 = pl.multiple_of((base + g * GB) * s * d // 128, 8)
            return flat_hbm.at[pl.ds(r0, rows_g)], vin.at[slot]

        def out_legs(g):
            b0 = pl.multiple_of(base + g * GB, 8)
            return vout, out_hbm.at[pl.ds(b0, GB)]

        def repack(slot):
            def mv(t, carry):
                w = t * 16
                r = w // 128
                c128 = w - r * 128
                x = vin[slot, r, pl.ds(c128, 16)]
                row = w // d
                c = w - row * d
                bb = row // s
                ss = row - bb * s
                vout[bb, ss, pl.ds(c, 16)] = x
                return carry

            lax.fori_loop(0, rows_g * 8, mv, 0)

        sr0, o0 = in_legs(0, 0)
        pltpu.async_copy(sr0, o0, isem.at[0])

        def body(i, carry):
            slot = jnp.bitwise_and(i, 1)

            @pl.when(i + 1 < ngr)
            def _():
                sr, o = in_legs(i + 1, 1 - slot)
                pltpu.async_copy(sr, o, isem.at[1 - slot])

            sr, o = in_legs(i, slot)
            pltpu.make_async_copy(sr, o, isem.at[slot]).wait()

            @pl.when(i > 0)
            def _():
                sr2, o2 = out_legs(i - 1)
                pltpu.make_async_copy(sr2, o2, osem).wait()

            repack(slot)
            sr2, o2 = out_legs(i)
            pltpu.async_copy(sr2, o2, osem)
            return carry

        lax.fori_loop(0, ngr, body, 0)
        srl, ol = out_legs(ngr - 1)
        pltpu.make_async_copy(srl, ol, osem).wait()

    return k(rows128)


def kernel(idx, weight):
    b, s = idx.shape
    v, d = weight.shape
    nch = (b * s) // (NW * CB)
    idx_flat = idx.reshape(-1).astype(jnp.int32)
    wlin = _compact(weight, v, d).reshape(v, d)
    out = _gather(idx_flat, wlin, nch, d)
    return _relayout(out.reshape(b * s * d // 128, 128), b, s, d)


# speedup vs baseline: 1.1558x; 1.1558x over previous
import functools

import jax
import jax.numpy as jnp
from jax import lax
from jax.experimental import pallas as pl
from jax.experimental.pallas import tpu as pltpu
from jax.experimental.pallas import tpu_sc as plsc

NC, NS = 2, 16
NW = NC * NS
CB = 128
NBUF = 8
LAG = 4

_MESH = plsc.VectorSubcoreMesh(core_axis_name="c", subcore_axis_name="s")


def _wid():
    return lax.axis_index("s") * NC + lax.axis_index("c")


GROW = 320


@functools.partial(jax.jit, static_argnums=(1, 2))
def _compact(table, v, d):
    ng = v // GROW
    tail = v - ng * GROW

    @functools.partial(
        pl.kernel,
        out_type=jax.ShapeDtypeStruct((v * d // 128, d * 2), jnp.float32),
        mesh=_MESH,
        scratch_types=[
            pltpu.VMEM((2, GROW, d), jnp.float32),
            pltpu.VMEM((2, GROW // 2, d * 2), jnp.float32),
            pltpu.SemaphoreType.DMA((2,)),
            pltpu.SemaphoreType.DMA((2,)),
        ],
        compiler_params=pltpu.CompilerParams(use_tc_tiling_on_sc=True),
    )
    def k(tab_hbm, out_hbm, vin, vout, isem, osem):
        wid = _wid()

        def in_legs(g, slot):
            r0 = pl.multiple_of(g * GROW, 8)
            return tab_hbm.at[pl.ds(r0, GROW)], vin.at[slot]

        def out_legs(g, slot):
            r0 = pl.multiple_of(g * (GROW // 2), 8)
            return vout.at[slot], out_hbm.at[pl.ds(r0, GROW // 2)]

        def repack(slot):
            def mv(t, carry):
                r = t >> 2
                c16 = jnp.bitwise_and(t, 3) << 4
                x = vin[slot, r, pl.ds(c16, 16)]
                vout[slot, r >> 1,
                     pl.ds(jnp.bitwise_and(r, 1) * d + c16, 16)] = x
                return carry

            lax.fori_loop(0, GROW * (d // 16), mv, 0, unroll=8)

        def body(i, carry):
            g = wid + i * NW
            slot = jnp.bitwise_and(i, 1)

            @pl.when(i > 1)
            def _():
                s, o = out_legs(g - 2 * NW, slot)
                pltpu.make_async_copy(s, o, osem.at[slot]).wait()

            s, o = in_legs(g, slot)
            pltpu.async_copy(s, o, isem.at[slot])
            pltpu.make_async_copy(s, o, isem.at[slot]).wait()
            repack(slot)
            s, o = out_legs(g, slot)
            pltpu.async_copy(s, o, osem.at[slot])
            return carry

        nmine = (ng - wid + NW - 1) // NW
        lax.fori_loop(0, nmine, body, 0)

        def drain(i, carry):
            j = nmine - 2 + i
            g = wid + j * NW

            @pl.when((j >= 0) & (g < ng))
            def _():
                s, o = out_legs(g, jnp.bitwise_and(j, 1))
                pltpu.make_async_copy(s, o, osem.at[jnp.bitwise_and(j, 1)]
                                      ).wait()
            return carry

        lax.fori_loop(0, 2, drain, 0)

        if tail:
            @pl.when(wid == 0)
            def _():
                r0 = ng * GROW
                s = tab_hbm.at[pl.ds(pl.multiple_of(r0, 8), tail)]
                o = vin.at[0, pl.ds(0, tail)]
                pltpu.async_copy(s, o, isem.at[0])
                pltpu.make_async_copy(s, o, isem.at[0]).wait()

                def mv(t, carry):
                    r = t >> 2
                    c16 = jnp.bitwise_and(t, 3) << 4
                    x = vin[0, r, pl.ds(c16, 16)]
                    vout[0, r >> 1,
                         pl.ds(jnp.bitwise_and(r, 1) * d + c16, 16)] = x
                    return carry

                lax.fori_loop(0, tail * (d // 16), mv, 0)
                s2 = vout.at[0, pl.ds(0, tail // 2)]
                o2 = out_hbm.at[pl.ds(pl.multiple_of(r0 // 2, 8), tail // 2)]
                pltpu.async_copy(s2, o2, osem.at[0])
                pltpu.make_async_copy(s2, o2, osem.at[0]).wait()

    return k(table)


@functools.partial(jax.jit, static_argnums=(2, 3))
def _gather(idx_flat, table, nch, d):
    rpw = nch * CB

    @functools.partial(
        pl.kernel,
        out_type=jax.ShapeDtypeStruct((NW * rpw, d), jnp.float32),
        mesh=_MESH,
        scratch_types=[
            pltpu.VMEM((nch * CB,), jnp.int32),
            pltpu.VMEM((NBUF, CB, d), jnp.float32),
            pltpu.SemaphoreType.DMA((NBUF,)),
            pltpu.SemaphoreType.DMA((NBUF,)),
        ],
        compiler_params=pltpu.CompilerParams(use_tc_tiling_on_sc=False),
    )
    def k(idx_hbm, table_hbm, out_hbm, idx_v, rows_v, gsem, osem):
        wid = _wid()
        base = wid * rpw
        pltpu.sync_copy(idx_hbm.at[pl.ds(base, rpw)], idx_v)

        def start_gather(j, slot):
            pltpu.async_copy(
                table_hbm.at[idx_v.at[pl.ds(j * CB, CB)]], rows_v.at[slot],
                gsem.at[slot])

        def drain_chunk(jd, slot):
            pltpu.make_async_copy(
                table_hbm.at[idx_v.at[pl.ds(jd * CB, CB)]], rows_v.at[slot],
                gsem.at[slot]
            ).wait()
            pltpu.async_copy(
                rows_v.at[slot], out_hbm.at[pl.ds(base + jd * CB, CB)],
                osem.at[slot])

        def wait_out(jd, slot):
            pltpu.make_async_copy(
                rows_v.at[slot], out_hbm.at[pl.ds(base + jd * CB, CB)],
                osem.at[slot]
            ).wait()

        for j in range(NBUF):
            start_gather(j, j)
            if j >= LAG:
                drain_chunk(j - LAG, j - LAG)

        def body(j, carry):
            slot = jnp.bitwise_and(j, NBUF - 1)
            wait_out(j - NBUF, slot)
            start_gather(j, slot)
            jw = j - LAG
            drain_chunk(jw, jnp.bitwise_and(jw, NBUF - 1))
            return carry

        lax.fori_loop(NBUF, nch, body, 0)

        for jd in range(nch - LAG, nch):
            drain_chunk(jd, jd % NBUF)
        for jd in range(nch - NBUF, nch):
            wait_out(jd, jd % NBUF)

    return k(idx_flat, table)


GB = 8


@functools.partial(jax.jit, static_argnums=(1, 2, 3))
def _relayout(rows128, b, s, d):
    bpw = b // NW
    ngr = bpw // GB
    rows_g = GB * s * d // 128

    @functools.partial(
        pl.kernel,
        out_type=jax.ShapeDtypeStruct((b, s, d), jnp.float32),
        mesh=_MESH,
        scratch_types=[
            pltpu.VMEM((2, GB * s * d // 128, 128), jnp.float32),
            pltpu.VMEM((GB, s, d), jnp.float32),
            pltpu.SemaphoreType.DMA((2,)),
            pltpu.SemaphoreType.DMA,
        ],
        compiler_params=pltpu.CompilerParams(use_tc_tiling_on_sc=True),
    )
    def k(flat_hbm, out_hbm, vin, vout, isem, osem):
        wid = _wid()
        base = wid * bpw

        def in_legs(g, slot):
            r0 = pl.multiple_of((base + g * GB) * s * d // 128, 8)
            return flat_hbm.at[pl.ds(r0, rows_g)], vin.at[slot]

        def out_legs(g):
            b0 = pl.multiple_of(base + g * GB, 8)
            return vout, out_hbm.at[pl.ds(b0, GB)]

        mpb = s * d // 16
        rpb = s * d // 128

        def repack(slot):
            def mvb(bb, carry):
                r0 = bb * rpb

                def mv(u, carry2):
                    x = vin[slot, r0 + (u >> 3),
                            pl.ds(jnp.bitwise_and(u, 7) << 4, 16)]
                    vout[bb, u >> 2,
                         pl.ds(jnp.bitwise_and(u, 3) << 4, 16)] = x
                    return carry2

                lax.fori_loop(0, mpb, mv, 0, unroll=8)
                return carry

            lax.fori_loop(0, GB, mvb, 0)

        sr0, o0 = in_legs(0, 0)
        pltpu.async_copy(sr0, o0, isem.at[0])

        def body(i, carry):
            slot = jnp.bitwise_and(i, 1)

            @pl.when(i + 1 < ngr)
            def _():
                sr, o = in_legs(i + 1, 1 - slot)
                pltpu.async_copy(sr, o, isem.at[1 - slot])

            sr, o = in_legs(i, slot)
            pltpu.make_async_copy(sr, o, isem.at[slot]).wait()

            @pl.when(i > 0)
            def _():
                sr2, o2 = out_legs(i - 1)
                pltpu.make_async_copy(sr2, o2, osem).wait()

            repack(slot)
            sr2, o2 = out_legs(i)
            pltpu.async_copy(sr2, o2, osem)
            return carry

        lax.fori_loop(0, ngr, body, 0)
        srl, ol = out_legs(ngr - 1)
        pltpu.make_async_copy(srl, ol, osem).wait()

    return k(rows128)


def kernel(idx, weight):
    b, s = idx.shape
    v, d = weight.shape
    nch = (b * s) // (NW * CB)
    idx_flat = idx.reshape(-1).astype(jnp.int32)
    wlin = _compact(weight, v, d).reshape(v, d)
    out = _gather(idx_flat, wlin, nch, d)
    return _relayout(out.reshape(b * s * d // 128, 128), b, s, d)


# speedup vs baseline: 1.7796x; 1.5397x over previous
import functools

import jax
import jax.numpy as jnp
from jax import lax
from jax.experimental import pallas as pl
from jax.experimental.pallas import tpu as pltpu
from jax.experimental.pallas import tpu_sc as plsc

NC, NS = 2, 16
NW = NC * NS
CB = 128
NBUF = 8
LAG = 4


@functools.partial(jax.jit, static_argnums=(2, 3))
def _gather(idx_w, table, nch, d):
    rpw = nch * CB
    mesh = plsc.VectorSubcoreMesh(core_axis_name="c", subcore_axis_name="s")

    @functools.partial(
        pl.kernel,
        out_type=jax.ShapeDtypeStruct((NW * rpw, d), jnp.float32),
        mesh=mesh,
        scratch_types=[
            pltpu.VMEM((nch, CB), jnp.int32),
            pltpu.VMEM((NBUF, CB, d), jnp.float32),
            pltpu.SemaphoreType.DMA((NBUF,)),
            pltpu.SemaphoreType.DMA((NBUF,)),
        ],
        compiler_params=pltpu.CompilerParams(use_tc_tiling_on_sc=False),
    )
    def k(idx_hbm, table_hbm, out_hbm, idx_v, rows_v, gsem, osem):
        wid = lax.axis_index("s") * NC + lax.axis_index("c")
        pltpu.sync_copy(idx_hbm.at[wid], idx_v)
        base = wid * rpw

        def start_gather(j, slot):
            pltpu.async_copy(
                table_hbm.at[idx_v.at[j]], rows_v.at[slot], gsem.at[slot])

        def drain_chunk(jd, slot):
            pltpu.make_async_copy(
                table_hbm.at[idx_v.at[jd]], rows_v.at[slot], gsem.at[slot]
            ).wait()
            pltpu.async_copy(
                rows_v.at[slot], out_hbm.at[pl.ds(base + jd * CB, CB)],
                osem.at[slot])

        def wait_out(jd, slot):
            pltpu.make_async_copy(
                rows_v.at[slot], out_hbm.at[pl.ds(base + jd * CB, CB)],
                osem.at[slot]
            ).wait()

        for j in range(NBUF):
            start_gather(j, j)
            if j >= LAG:
                drain_chunk(j - LAG, j - LAG)

        def body(j, carry):
            slot = jnp.bitwise_and(j, NBUF - 1)
            jd = j - NBUF
            wait_out(jd, slot)
            start_gather(j, slot)
            jw = j - LAG
            drain_chunk(jw, jnp.bitwise_and(jw, NBUF - 1))
            return carry

        lax.fori_loop(NBUF, nch, body, 0)

        for jd in range(nch - LAG, nch):
            drain_chunk(jd, jd % NBUF)
        for jd in range(nch - NBUF, nch):
            wait_out(jd, jd % NBUF)

    return k(idx_w, table)


def kernel(idx, weight):
    b = idx.size
    d = weight.shape[-1]
    nch = b // (NW * CB)
    idx_w = idx.reshape(NW, nch, CB).astype(jnp.int32)
    out = _gather(idx_w, weight, nch, d)
    return out.reshape(idx.shape + (d,))
